# baseline (speedup 1.0000x reference)
import functools

import numpy as np
import jax
import jax.numpy as jnp
from jax import lax
from jax.experimental import pallas as pl
from jax.experimental.pallas import tpu as pltpu
from jax.experimental.pallas import tpu_sc as plsc

N_V = 100000
NP = 100352
N_F = 200000
NTILES = 32
N_BULK = 199936
N_TAIL = N_F - N_BULK
FPT = N_BULK // NTILES
CHUNK = 1024
NFULL = 6
REM = FPT - NFULL * CHUNK
CPT = NP // 16
RW = 8

_COLS = ((0, (1, 2)), (1, (0, 2)), (2, (1, 0)))


def _sc_scatter(vert_pad, zeros, faces_t, faces_tail):
    mesh = plsc.VectorSubcoreMesh(core_axis_name="c", subcore_axis_name="s")

    @functools.partial(
        pl.kernel,
        mesh=mesh,
        compiler_params=pltpu.CompilerParams(use_tc_tiling_on_sc=False),
        out_type=jax.ShapeDtypeStruct((2, NP, RW), jnp.float32),
        scratch_types=[
            pltpu.VMEM_SHARED((NP, RW), jnp.float32),
            [[pltpu.VMEM((CHUNK,), jnp.int32)] * 3] * 2,
            [[pltpu.VMEM((CHUNK, RW), jnp.float32)] * 3] * 2,
            [pltpu.VMEM((REM,), jnp.int32)] * 3,
            pltpu.VMEM((REM, RW), jnp.float32),
            [pltpu.VMEM((N_TAIL,), jnp.int32)] * 3,
            pltpu.VMEM((N_TAIL, RW), jnp.float32),
            [pltpu.SemaphoreType.DMA] * 2,
            [pltpu.SemaphoreType.DMA] * 2,
            [pltpu.SemaphoreType.DMA] * 2,
        ],
    )
    def body(vp_hbm, z_hbm, ft_hbm, tail_hbm, out_hbm,
             acc_sh, colb, rowsb, colr, rowsr, colt, rowst,
             isem, gsem, ssem):
        cid = lax.axis_index("c")
        sid = lax.axis_index("s")
        wid = sid * 2 + cid
        r0 = sid * CPT
        fbase = wid * FPT

        pltpu.sync_copy(z_hbm.at[pl.ds(r0, CPT)], acc_sh.at[pl.ds(r0, CPT)])
        plsc.subcore_barrier()

        def edge_pipeline(nfull, rem, colr_, rowsr_):
            def start_idx(c):
                b = c % 2
                f0 = fbase + c * CHUNK
                return [pltpu.async_copy(ft_hbm.at[col, pl.ds(f0, CHUNK)],
                                         colb[b][col], isem[b])
                        for col in range(3)]

            idx_h = {0: start_idx(0)}
            gat_h = {}
            sca_h = {}
            for c in range(nfull):
                b = c % 2
                for h in idx_h.pop(c):
                    h.wait()
                gat_h[c] = [pltpu.async_copy(vp_hbm.at[colb[b][cs]],
                                             rowsb[b][cs], gsem[b])
                            for cs, _ in _COLS]
                if c + 1 < nfull:
                    if c >= 1:
                        for h in sca_h.pop(c - 1):
                            h.wait()
                    idx_h[c + 1] = start_idx(c + 1)
                sca_h[c] = []
                for k, (cs, (cd0, cd1)) in enumerate(_COLS):
                    gat_h[c][k].wait()
                    sca_h[c].append(pltpu.async_copy(
                        rowsb[b][cs], acc_sh.at[colb[b][cd0]], ssem[b],
                        add=True))
                    sca_h[c].append(pltpu.async_copy(
                        rowsb[b][cs], acc_sh.at[colb[b][cd1]], ssem[b],
                        add=True))
            for c in (nfull - 2, nfull - 1):
                for h in sca_h.pop(c):
                    h.wait()

            f0 = fbase + nfull * CHUNK
            for col in range(3):
                pltpu.sync_copy(ft_hbm.at[col, pl.ds(f0, rem)], colr_[col])
            for cs, (cd0, cd1) in _COLS:
                pltpu.sync_copy(vp_hbm.at[colr_[cs]], rowsr_)
                pltpu.sync_copy(rowsr_, acc_sh.at[colr_[cd0]], add=True)
                pltpu.sync_copy(rowsr_, acc_sh.at[colr_[cd1]], add=True)

        edge_pipeline(NFULL, REM, colr, rowsr)

        @pl.when(wid == NTILES - 1)
        def _():
            for col in range(3):
                pltpu.sync_copy(tail_hbm.at[col], colt[col])
            for cs, (cd0, cd1) in _COLS:
                pltpu.sync_copy(vp_hbm.at[colt[cs]], rowst)
                pltpu.sync_copy(rowst, acc_sh.at[colt[cd0]], add=True)
                pltpu.sync_copy(rowst, acc_sh.at[colt[cd1]], add=True)

        plsc.subcore_barrier()
        pltpu.sync_copy(acc_sh.at[pl.ds(r0, CPT)],
                        out_hbm.at[cid, pl.ds(r0, CPT)])

    return body(vert_pad, zeros, faces_t, faces_tail)


def _finalize_body(p, v, tdeg, tsum, o):
    x = p[0] + p[1]
    deg = jnp.maximum(
        jnp.dot(x, tdeg[...], preferred_element_type=jnp.float32,
                precision=lax.Precision.HIGHEST), 1.0)
    lap = x / deg - v[...]
    sq = lap * lap
    o[...] = jnp.sqrt(
        jnp.dot(sq, tsum[...], preferred_element_type=jnp.float32,
                precision=lax.Precision.HIGHEST))


def kernel(vert, faces):
    fb = faces[:N_BULK].reshape(N_BULK // 128, 384).astype(jnp.float32)
    w384 = np.arange(384)
    csel = [
        jnp.asarray((w384[:, None] == 3 * np.arange(128)[None, :] + c)
                    .astype(np.float32))
        for c in range(3)
    ]
    faces_t = jnp.stack(
        [jnp.dot(fb, csel[c], precision=lax.Precision.HIGHEST)
         .astype(jnp.int32) for c in range(3)]).reshape(3, N_BULK)
    faces_tail = faces[N_BULK:].T

    nr = NP * RW // 128
    lanes = np.arange(128)
    psel = jnp.asarray(
        ((lanes[None, :] % 8 < 3)
         & (3 * (lanes[None, :] // 8) + lanes[None, :] % 8
            == np.arange(48)[:, None])).astype(np.float32))
    ones_lane = jnp.asarray((lanes % 8 == 3).astype(np.float32))
    vp128 = jnp.dot(vert.reshape(N_V // 16, 48), psel,
                    precision=lax.Precision.HIGHEST) + ones_lane[None, :]
    vp128 = jnp.concatenate(
        [vp128, jnp.zeros((nr - N_V // 16, 128), jnp.float32)], axis=0)
    vert_pad = vp128.reshape(NP, RW)
    zeros = jnp.zeros((NP, RW), jnp.float32)

    part = _sc_scatter(vert_pad, zeros, faces_t, faces_tail)

    tdeg = jnp.asarray(
        (lanes[:, None] == 8 * (lanes[None, :] // 8) + 3).astype(np.float32))
    tsum = jnp.asarray(
        ((lanes[:, None] // 8 == np.arange(16)[None, :])
         & (lanes[:, None] % 8 < 3)).astype(np.float32))
    curve = pl.pallas_call(
        _finalize_body,
        out_shape=jax.ShapeDtypeStruct((nr, 16), jnp.float32),
    )(part.reshape(2, nr, 128), vp128, tdeg, tsum)
    return curve.reshape(NP)[:N_V]


# speedup vs baseline: 1.8792x; 1.8792x over previous
import functools

import numpy as np
import jax
import jax.numpy as jnp
from jax import lax
from jax.experimental import pallas as pl
from jax.experimental.pallas import tpu as pltpu
from jax.experimental.pallas import tpu_sc as plsc

N_V = 100000
NP = 100352
N_F = 200000
NTILES = 32
FPT = 6256
LAST = N_F - (NTILES - 1) * FPT
CHUNK = 1024
NFULL = 6
REM = FPT - NFULL * CHUNK
REML = LAST - 5 * CHUNK
CPT = NP // 16
RW = 8

_COLS = ((0, (1, 2)), (1, (0, 2)), (2, (1, 0)))


def _sc_scatter(vert_pad, zeros, faces_t):
    mesh = plsc.VectorSubcoreMesh(core_axis_name="c", subcore_axis_name="s")

    @functools.partial(
        pl.kernel,
        mesh=mesh,
        compiler_params=pltpu.CompilerParams(use_tc_tiling_on_sc=False),
        out_type=jax.ShapeDtypeStruct((2, NP, RW), jnp.float32),
        scratch_types=[
            pltpu.VMEM_SHARED((NP, RW), jnp.float32),
            [[pltpu.VMEM((CHUNK,), jnp.int32)] * 3] * 2,
            [[pltpu.VMEM((CHUNK, RW), jnp.float32)] * 3] * 2,
            [pltpu.VMEM((REM,), jnp.int32)] * 3,
            pltpu.VMEM((REM, RW), jnp.float32),
            [pltpu.VMEM((REML,), jnp.int32)] * 3,
            pltpu.VMEM((REML, RW), jnp.float32),
            [pltpu.SemaphoreType.DMA] * 2,
            [pltpu.SemaphoreType.DMA] * 2,
            [pltpu.SemaphoreType.DMA] * 2,
        ],
    )
    def body(vp_hbm, z_hbm, ft_hbm, out_hbm,
             acc_sh, colb, rowsb, colr, rowsr, colrl, rowsrl,
             isem, gsem, ssem):
        cid = lax.axis_index("c")
        sid = lax.axis_index("s")
        wid = sid * 2 + cid
        r0 = sid * CPT
        fbase = wid * FPT

        pltpu.sync_copy(z_hbm.at[pl.ds(r0, CPT)], acc_sh.at[pl.ds(r0, CPT)])
        plsc.subcore_barrier()

        def edge_pipeline(nfull, rem, colr_, rowsr_):
            def start_idx(c):
                b = c % 2
                f0 = fbase + c * CHUNK
                return [pltpu.async_copy(ft_hbm.at[col, pl.ds(f0, CHUNK)],
                                         colb[b][col], isem[b])
                        for col in range(3)]

            idx_h = {0: start_idx(0)}
            gat_h = {}
            sca_h = {}
            for c in range(nfull):
                b = c % 2
                for h in idx_h.pop(c):
                    h.wait()
                gat_h[c] = [pltpu.async_copy(vp_hbm.at[colb[b][cs]],
                                             rowsb[b][cs], gsem[b])
                            for cs, _ in _COLS]
                if c + 1 < nfull:
                    if c >= 1:
                        for h in sca_h.pop(c - 1):
                            h.wait()
                    idx_h[c + 1] = start_idx(c + 1)
                sca_h[c] = []
                for k, (cs, (cd0, cd1)) in enumerate(_COLS):
                    gat_h[c][k].wait()
                    sca_h[c].append(pltpu.async_copy(
                        rowsb[b][cs], acc_sh.at[colb[b][cd0]], ssem[b],
                        add=True))
                    sca_h[c].append(pltpu.async_copy(
                        rowsb[b][cs], acc_sh.at[colb[b][cd1]], ssem[b],
                        add=True))
            for c in (nfull - 2, nfull - 1):
                for h in sca_h.pop(c):
                    h.wait()

            f0 = fbase + nfull * CHUNK
            for col in range(3):
                pltpu.sync_copy(ft_hbm.at[col, pl.ds(f0, rem)], colr_[col])
            for cs, (cd0, cd1) in _COLS:
                pltpu.sync_copy(vp_hbm.at[colr_[cs]], rowsr_)
                pltpu.sync_copy(rowsr_, acc_sh.at[colr_[cd0]], add=True)
                pltpu.sync_copy(rowsr_, acc_sh.at[colr_[cd1]], add=True)

        @pl.when(wid < NTILES - 1)
        def _():
            edge_pipeline(NFULL, REM, colr, rowsr)

        @pl.when(wid == NTILES - 1)
        def _():
            edge_pipeline(5, REML, colrl, rowsrl)

        plsc.subcore_barrier()
        pltpu.sync_copy(acc_sh.at[pl.ds(r0, CPT)],
                        out_hbm.at[cid, pl.ds(r0, CPT)])

    return body(vert_pad, zeros, faces_t)


def _finalize_body(p, v, tdeg, tsum, o):
    x = p[0] + p[1]
    deg = jnp.maximum(
        jnp.dot(x, tdeg[...], preferred_element_type=jnp.float32,
                precision=lax.Precision.HIGHEST), 1.0)
    lap = x / deg - v[...]
    sq = lap * lap
    o[...] = jnp.sqrt(
        jnp.dot(sq, tsum[...], preferred_element_type=jnp.float32,
                precision=lax.Precision.HIGHEST))


def kernel(vert, faces):
    faces_t = faces.T

    nr = NP * RW // 128
    lanes = np.arange(128)
    psel = jnp.asarray(
        ((lanes[None, :] % 8 < 3)
         & (3 * (lanes[None, :] // 8) + lanes[None, :] % 8
            == np.arange(48)[:, None])).astype(np.float32))
    ones_lane = jnp.asarray((lanes % 8 == 3).astype(np.float32))
    vp128 = jnp.dot(vert.reshape(N_V // 16, 48), psel,
                    precision=lax.Precision.HIGHEST) + ones_lane[None, :]
    vp128 = jnp.concatenate(
        [vp128, jnp.zeros((nr - N_V // 16, 128), jnp.float32)], axis=0)
    vert_pad = vp128.reshape(NP, RW)
    zeros = jnp.zeros((NP, RW), jnp.float32)

    part = _sc_scatter(vert_pad, zeros, faces_t)

    tdeg = jnp.asarray(
        (lanes[:, None] == 8 * (lanes[None, :] // 8) + 3).astype(np.float32))
    tsum = jnp.asarray(
        ((lanes[:, None] // 8 == np.arange(16)[None, :])
         & (lanes[:, None] % 8 < 3)).astype(np.float32))
    curve = pl.pallas_call(
        _finalize_body,
        out_shape=jax.ShapeDtypeStruct((nr, 16), jnp.float32),
    )(part.reshape(2, nr, 128), vp128, tdeg, tsum)
    return curve.reshape(NP)[:N_V]
